# Initial kernel scaffold; baseline (speedup 1.0000x reference)
import jax
import jax.numpy as jnp
from jax.experimental import pallas as pl


def kernel(x, edge_index, batch, Wa0, ba0, Wb0, bb0, g0, beta0, Wa1, ba1, Wb1, bb1, g1, beta1, Wa2, ba2, Wb2, bb2, g2, beta2):
    raise NotImplementedError("write your pallas kernel here")


# speedup vs baseline: 8.9115x; 8.9115x over previous
import functools

import jax
import jax.numpy as jnp
from jax import lax
from jax.experimental import pallas as pl
from jax.experimental.pallas import tpu as pltpu
from jax.experimental.pallas import tpu_sc as plsc

N = 10000
D = 128
BM = 400
GRID_M = N // BM
NPAD = 10400
PBLK = NPAD // BM
CH = 128
NSUB = 16
NW = 2 * NSUB
RSL = 632
RSLW = 656


def _bf16_dot(a, b):
    return jnp.dot(a.astype(jnp.bfloat16), b.astype(jnp.bfloat16),
                   preferred_element_type=jnp.float32)


def _mm0_body(x_ref, w_ref, y_ref):
    y_ref[...] = jnp.dot(x_ref[...], w_ref[...].astype(jnp.float32),
                         preferred_element_type=jnp.float32,
                         precision=lax.Precision.HIGHEST)


def _matmul0(x, w):
    kin = x.shape[1]
    return pl.pallas_call(
        _mm0_body,
        grid=(GRID_M,),
        in_specs=[
            pl.BlockSpec((BM, kin), lambda i: (i, 0)),
            pl.BlockSpec((kin, D), lambda i: (0, 0)),
        ],
        out_specs=pl.BlockSpec((BM, D), lambda i: (i, 0)),
        out_shape=jax.ShapeDtypeStruct((N, D), jnp.float32),
    )(x, w)


def _sc_segsum(z, src3, dst3):
    nch = src3.shape[1]
    mesh = plsc.VectorSubcoreMesh(core_axis_name="c", subcore_axis_name="s")

    @functools.partial(
        pl.kernel,
        out_type=jax.ShapeDtypeStruct((2 * NPAD, D), jnp.float32),
        mesh=mesh,
        scratch_types=[
            pltpu.VMEM((nch, CH), jnp.int32),
            pltpu.VMEM((nch, CH), jnp.int32),
            pltpu.VMEM((CH, D), jnp.float32),
            pltpu.VMEM_SHARED((NPAD, D), jnp.float32),
            pltpu.SemaphoreType.DMA,
        ],
    )
    def k(z_hbm, src_hbm, dst_hbm, p_hbm, src_v, dst_v, rows_v, acc_sh, sem):
        c = lax.axis_index("c")
        s = lax.axis_index("s")
        w = c * NSUB + s

        ist = pl.multiple_of(jnp.minimum(s * RSL, N - RSL), 8)
        pltpu.sync_copy(z_hbm.at[pl.ds(ist, RSL)],
                        acc_sh.at[pl.ds(ist, RSL)])
        pltpu.sync_copy(src_hbm.at[w], src_v)
        pltpu.sync_copy(dst_hbm.at[w], dst_v)
        plsc.subcore_barrier()

        @pl.loop(0, nch)
        def _(j):
            pltpu.async_copy(z_hbm.at[src_v.at[j]], rows_v, sem).wait()
            pltpu.sync_copy(rows_v, acc_sh.at[dst_v.at[j]], add=True)

        plsc.subcore_barrier()
        wst = pl.multiple_of(jnp.minimum(s * RSLW, NPAD - RSLW), 8)
        off = pl.multiple_of(c * NPAD + wst, 8)
        pltpu.sync_copy(acc_sh.at[pl.ds(wst, RSLW)], p_hbm.at[pl.ds(off, RSLW)])

    return k(z, src3, dst3)


def _mlp0_body(p0_ref, p1_ref, z_ref, ba_ref, wb_ref, bb_ref,
               r_ref, s1_ref, s2_ref):
    i = pl.program_id(0)
    a = p0_ref[...] + p1_ref[...] - z_ref[...]
    m = jnp.maximum(a + ba_ref[...], 0.0)
    t = _bf16_dot(m, wb_ref[...]) + bb_ref[...]
    r = jnp.maximum(t, 0.0)
    r_ref[...] = r

    @pl.when(i == 0)
    def _():
        s1_ref[...] = jnp.zeros_like(s1_ref)
        s2_ref[...] = jnp.zeros_like(s2_ref)

    s1_ref[...] += jnp.sum(r, axis=0, keepdims=True)
    s2_ref[...] += jnp.sum(r * r, axis=0, keepdims=True)


def _mlp_body(p0_ref, p1_ref, z_ref, wa_ref, ba_ref, wb_ref, bb_ref,
              r_ref, s1_ref, s2_ref):
    i = pl.program_id(0)
    a = p0_ref[...] + p1_ref[...] - z_ref[...]
    m = _bf16_dot(a, wa_ref[...])
    m = jnp.maximum(m + ba_ref[...], 0.0)
    t = _bf16_dot(m, wb_ref[...]) + bb_ref[...]
    r = jnp.maximum(t, 0.0)
    r_ref[...] = r

    @pl.when(i == 0)
    def _():
        s1_ref[...] = jnp.zeros_like(s1_ref)
        s2_ref[...] = jnp.zeros_like(s2_ref)

    s1_ref[...] += jnp.sum(r, axis=0, keepdims=True)
    s2_ref[...] += jnp.sum(r * r, axis=0, keepdims=True)


_ROW_SPEC = pl.BlockSpec((BM, D), lambda i: (i, 0))
_P0_SPEC = pl.BlockSpec((BM, D), lambda i: (i, 0))
_P1_SPEC = pl.BlockSpec((BM, D), lambda i: (i + PBLK, 0))
_VEC_SPEC = pl.BlockSpec((1, D), lambda i: (0, 0))
_MAT_SPEC = pl.BlockSpec((D, D), lambda i: (0, 0))
_STAT_OUT = [
    pl.BlockSpec((BM, D), lambda i: (i, 0)),
    pl.BlockSpec((1, D), lambda i: (0, 0)),
    pl.BlockSpec((1, D), lambda i: (0, 0)),
]
_STAT_SHAPE = [
    jax.ShapeDtypeStruct((N, D), jnp.float32),
    jax.ShapeDtypeStruct((1, D), jnp.float32),
    jax.ShapeDtypeStruct((1, D), jnp.float32),
]


def _layer_mlp0(p, z, ba, wb, bb):
    return pl.pallas_call(
        _mlp0_body,
        grid=(GRID_M,),
        in_specs=[_P0_SPEC, _P1_SPEC, _ROW_SPEC, _VEC_SPEC, _MAT_SPEC,
                  _VEC_SPEC],
        out_specs=_STAT_OUT,
        out_shape=_STAT_SHAPE,
    )(p, p, z, ba.reshape(1, D), wb, bb.reshape(1, D))


def _layer_mlp(p, z, wa, ba, wb, bb):
    return pl.pallas_call(
        _mlp_body,
        grid=(GRID_M,),
        in_specs=[_P0_SPEC, _P1_SPEC, _ROW_SPEC, _MAT_SPEC, _VEC_SPEC,
                  _MAT_SPEC, _VEC_SPEC],
        out_specs=_STAT_OUT,
        out_shape=_STAT_SHAPE,
    )(p, p, z, wa, ba.reshape(1, D), wb, bb.reshape(1, D))


def _bn_body(r_ref, s1_ref, s2_ref, g_ref, b_ref, h_ref):
    mu = s1_ref[...] * (1.0 / N)
    var = s2_ref[...] * (1.0 / N) - mu * mu
    scale = lax.rsqrt(var + 1e-5) * g_ref[...]
    h_ref[...] = (r_ref[...] - mu) * scale + b_ref[...]


def _bn_only(r, s1, s2, g, beta):
    return pl.pallas_call(
        _bn_body,
        grid=(GRID_M,),
        in_specs=[_ROW_SPEC, _VEC_SPEC, _VEC_SPEC, _VEC_SPEC, _VEC_SPEC],
        out_specs=pl.BlockSpec((BM, D), lambda i: (i, 0)),
        out_shape=jax.ShapeDtypeStruct((N, D), jnp.float32),
    )(r, s1, s2, g.reshape(1, D), beta.reshape(1, D))


def kernel(x, edge_index, batch, Wa0, ba0, Wb0, bb0, g0, beta0,
           Wa1, ba1, Wb1, bb1, g1, beta1, Wa2, ba2, Wb2, bb2, g2, beta2):
    src = edge_index[0]
    dst = edge_index[1]
    e = src.shape[0]
    n_per_w = -(-e // (NW * CH)) * CH
    pad = NW * n_per_w - e
    src_p = jnp.concatenate([src, jnp.zeros((pad,), jnp.int32)])
    dst_p = jnp.concatenate([dst, jnp.full((pad,), N, jnp.int32)])
    src3 = src_p.reshape(NW, n_per_w // CH, CH)
    dst3 = dst_p.reshape(NW, n_per_w // CH, CH)

    y0 = _matmul0(x, Wa0.astype(jnp.bfloat16))
    p = _sc_segsum(y0, src3, dst3)
    r, s1, s2 = _layer_mlp0(p, y0, ba0, Wb0, bb0)
    h = _bn_only(r, s1, s2, g0, beta0)
    outs = [h]

    for wa, ba, wb, bb, g, beta in ((Wa1, ba1, Wb1, bb1, g1, beta1),
                                    (Wa2, ba2, Wb2, bb2, g2, beta2)):
        p = _sc_segsum(h, src3, dst3)
        r, s1, s2 = _layer_mlp(p, h, wa, ba, wb, bb)
        h = _bn_only(r, s1, s2, g, beta)
        outs.append(h)

    rep = jnp.concatenate(outs, axis=1)
    return rep[::7]


# speedup vs baseline: 9.5116x; 1.0673x over previous
import functools

import jax
import jax.numpy as jnp
from jax import lax
from jax.experimental import pallas as pl
from jax.experimental.pallas import tpu as pltpu
from jax.experimental.pallas import tpu_sc as plsc

N = 10000
D = 128
BM = 400
GRID_M = N // BM
NPAD = 10400
PBLK = NPAD // BM
CH = 128
NSUB = 16
NW = 2 * NSUB
RSL = 632
RSLW = 656


def _bf16_dot(a, b):
    return jnp.dot(a.astype(jnp.bfloat16), b.astype(jnp.bfloat16),
                   preferred_element_type=jnp.float32)


def _mm0_body(x_ref, w_ref, y_ref):
    y_ref[...] = jnp.dot(x_ref[...], w_ref[...].astype(jnp.float32),
                         preferred_element_type=jnp.float32,
                         precision=lax.Precision.HIGHEST)


def _matmul0(x, w):
    kin = x.shape[1]
    return pl.pallas_call(
        _mm0_body,
        grid=(GRID_M,),
        in_specs=[
            pl.BlockSpec((BM, kin), lambda i: (i, 0)),
            pl.BlockSpec((kin, D), lambda i: (0, 0)),
        ],
        out_specs=pl.BlockSpec((BM, D), lambda i: (i, 0)),
        out_shape=jax.ShapeDtypeStruct((N, D), jnp.float32),
    )(x, w)


def _sc_segsum(z, src3, dst3):
    nch = src3.shape[1]
    mesh = plsc.VectorSubcoreMesh(core_axis_name="c", subcore_axis_name="s")

    @functools.partial(
        pl.kernel,
        out_type=jax.ShapeDtypeStruct((2 * NPAD, D), jnp.float32),
        mesh=mesh,
        scratch_types=[
            pltpu.VMEM((nch, CH), jnp.int32),
            pltpu.VMEM((nch, CH), jnp.int32),
            pltpu.VMEM((CH, D), jnp.float32),
            pltpu.VMEM((CH, D), jnp.float32),
            pltpu.VMEM_SHARED((NPAD, D), jnp.float32),
            pltpu.SemaphoreType.DMA,
            pltpu.SemaphoreType.DMA,
        ],
    )
    def k(z_hbm, src_hbm, dst_hbm, p_hbm, src_v, dst_v, rows_v0, rows_v1,
          acc_sh, sem0, sem1):
        c = lax.axis_index("c")
        s = lax.axis_index("s")
        w = c * NSUB + s

        ist = pl.multiple_of(jnp.minimum(s * RSL, N - RSL), 8)
        pltpu.sync_copy(z_hbm.at[pl.ds(ist, RSL)],
                        acc_sh.at[pl.ds(ist, RSL)])
        pltpu.sync_copy(src_hbm.at[w], src_v)
        pltpu.sync_copy(dst_hbm.at[w], dst_v)
        plsc.subcore_barrier()

        bufs = (rows_v0, rows_v1)
        sems = (sem0, sem1)
        pltpu.async_copy(z_hbm.at[src_v.at[0]], rows_v0, sem0)
        pltpu.async_copy(z_hbm.at[src_v.at[1]], rows_v1, sem1)

        @pl.loop(0, nch, step=2)
        def _(j):
            for b in range(2):
                pltpu.make_async_copy(z_hbm.at[src_v.at[j + b]],
                                      bufs[b], sems[b]).wait()
                pltpu.sync_copy(bufs[b], acc_sh.at[dst_v.at[j + b]], add=True)

                @pl.when(j + b + 2 < nch)
                def _():
                    pltpu.async_copy(z_hbm.at[src_v.at[j + b + 2]],
                                     bufs[b], sems[b])

        plsc.subcore_barrier()
        wst = pl.multiple_of(jnp.minimum(s * RSLW, NPAD - RSLW), 8)
        off = pl.multiple_of(c * NPAD + wst, 8)
        pltpu.sync_copy(acc_sh.at[pl.ds(wst, RSLW)], p_hbm.at[pl.ds(off, RSLW)])

    return k(z, src3, dst3)


def _mlp0_body(p0_ref, p1_ref, z_ref, ba_ref, wb_ref, bb_ref,
               r_ref, s1_ref, s2_ref):
    i = pl.program_id(0)
    a = p0_ref[...] + p1_ref[...] - z_ref[...]
    m = jnp.maximum(a + ba_ref[...], 0.0)
    t = _bf16_dot(m, wb_ref[...]) + bb_ref[...]
    r = jnp.maximum(t, 0.0)
    r_ref[...] = r

    @pl.when(i == 0)
    def _():
        s1_ref[...] = jnp.zeros_like(s1_ref)
        s2_ref[...] = jnp.zeros_like(s2_ref)

    s1_ref[...] += jnp.sum(r, axis=0, keepdims=True)
    s2_ref[...] += jnp.sum(r * r, axis=0, keepdims=True)


def _mlp_body(p0_ref, p1_ref, z_ref, wa_ref, ba_ref, wb_ref, bb_ref,
              r_ref, s1_ref, s2_ref):
    i = pl.program_id(0)
    a = p0_ref[...] + p1_ref[...] - z_ref[...]
    m = _bf16_dot(a, wa_ref[...])
    m = jnp.maximum(m + ba_ref[...], 0.0)
    t = _bf16_dot(m, wb_ref[...]) + bb_ref[...]
    r = jnp.maximum(t, 0.0)
    r_ref[...] = r

    @pl.when(i == 0)
    def _():
        s1_ref[...] = jnp.zeros_like(s1_ref)
        s2_ref[...] = jnp.zeros_like(s2_ref)

    s1_ref[...] += jnp.sum(r, axis=0, keepdims=True)
    s2_ref[...] += jnp.sum(r * r, axis=0, keepdims=True)


_ROW_SPEC = pl.BlockSpec((BM, D), lambda i: (i, 0))
_P0_SPEC = pl.BlockSpec((BM, D), lambda i: (i, 0))
_P1_SPEC = pl.BlockSpec((BM, D), lambda i: (i + PBLK, 0))
_VEC_SPEC = pl.BlockSpec((1, D), lambda i: (0, 0))
_MAT_SPEC = pl.BlockSpec((D, D), lambda i: (0, 0))
_STAT_OUT = [
    pl.BlockSpec((BM, D), lambda i: (i, 0)),
    pl.BlockSpec((1, D), lambda i: (0, 0)),
    pl.BlockSpec((1, D), lambda i: (0, 0)),
]
_STAT_SHAPE = [
    jax.ShapeDtypeStruct((N, D), jnp.float32),
    jax.ShapeDtypeStruct((1, D), jnp.float32),
    jax.ShapeDtypeStruct((1, D), jnp.float32),
]


def _layer_mlp0(p, z, ba, wb, bb):
    return pl.pallas_call(
        _mlp0_body,
        grid=(GRID_M,),
        in_specs=[_P0_SPEC, _P1_SPEC, _ROW_SPEC, _VEC_SPEC, _MAT_SPEC,
                  _VEC_SPEC],
        out_specs=_STAT_OUT,
        out_shape=_STAT_SHAPE,
    )(p, p, z, ba.reshape(1, D), wb, bb.reshape(1, D))


def _layer_mlp(p, z, wa, ba, wb, bb):
    return pl.pallas_call(
        _mlp_body,
        grid=(GRID_M,),
        in_specs=[_P0_SPEC, _P1_SPEC, _ROW_SPEC, _MAT_SPEC, _VEC_SPEC,
                  _MAT_SPEC, _VEC_SPEC],
        out_specs=_STAT_OUT,
        out_shape=_STAT_SHAPE,
    )(p, p, z, wa, ba.reshape(1, D), wb, bb.reshape(1, D))


def _bn_body(r_ref, s1_ref, s2_ref, g_ref, b_ref, h_ref):
    mu = s1_ref[...] * (1.0 / N)
    var = s2_ref[...] * (1.0 / N) - mu * mu
    scale = lax.rsqrt(var + 1e-5) * g_ref[...]
    h_ref[...] = (r_ref[...] - mu) * scale + b_ref[...]


def _bn_only(r, s1, s2, g, beta):
    return pl.pallas_call(
        _bn_body,
        grid=(GRID_M,),
        in_specs=[_ROW_SPEC, _VEC_SPEC, _VEC_SPEC, _VEC_SPEC, _VEC_SPEC],
        out_specs=pl.BlockSpec((BM, D), lambda i: (i, 0)),
        out_shape=jax.ShapeDtypeStruct((N, D), jnp.float32),
    )(r, s1, s2, g.reshape(1, D), beta.reshape(1, D))


def kernel(x, edge_index, batch, Wa0, ba0, Wb0, bb0, g0, beta0,
           Wa1, ba1, Wb1, bb1, g1, beta1, Wa2, ba2, Wb2, bb2, g2, beta2):
    src = edge_index[0]
    dst = edge_index[1]
    e = src.shape[0]
    n_per_w = -(-e // (NW * CH)) * CH
    pad = NW * n_per_w - e
    src_p = jnp.concatenate([src, jnp.zeros((pad,), jnp.int32)])
    dst_p = jnp.concatenate([dst, jnp.full((pad,), N, jnp.int32)])
    src3 = src_p.reshape(NW, n_per_w // CH, CH)
    dst3 = dst_p.reshape(NW, n_per_w // CH, CH)

    y0 = _matmul0(x, Wa0.astype(jnp.bfloat16))
    p = _sc_segsum(y0, src3, dst3)
    r, s1, s2 = _layer_mlp0(p, y0, ba0, Wb0, bb0)
    h = _bn_only(r, s1, s2, g0, beta0)
    outs = [h]

    for wa, ba, wb, bb, g, beta in ((Wa1, ba1, Wb1, bb1, g1, beta1),
                                    (Wa2, ba2, Wb2, bb2, g2, beta2)):
        p = _sc_segsum(h, src3, dst3)
        r, s1, s2 = _layer_mlp(p, h, wa, ba, wb, bb)
        h = _bn_only(r, s1, s2, g, beta)
        outs.append(h)

    rep = jnp.concatenate(outs, axis=1)
    return rep[::7]
